# baseline (speedup 1.0000x reference)
import functools

import jax
import jax.numpy as jnp
from jax.experimental import pallas as pl

ALPHA = 1.702
LIMIT = 7.0


def _moe_body(hs_ref, ri_ref, rw_ref, wg_ref, wu_ref, bg_ref, bu_ref,
              wd_ref, bd_ref, out_ref):
    e = pl.program_id(0)
    hs = hs_ref[...]
    ri = ri_ref[...]
    rw = rw_ref[...]

    T, E = rw.shape
    lane = jax.lax.broadcasted_iota(jnp.int32, (T, E), 1)
    wt = jnp.sum(jnp.where(lane == ri, rw, 0.0), axis=1, keepdims=True)
    w_col = jnp.where(ri == e, wt, 0.0)

    gate = jnp.dot(hs, wg_ref[0], preferred_element_type=jnp.float32) + bg_ref[0]
    up = jnp.dot(hs, wu_ref[0], preferred_element_type=jnp.float32) + bu_ref[0]
    gate = jnp.minimum(gate, LIMIT)
    up = jnp.clip(up, -LIMIT, LIMIT)
    glu = gate * jax.nn.sigmoid(gate * ALPHA)
    fused = (up + 1.0) * glu

    contrib = jnp.dot(w_col * fused, wd_ref[0],
                      preferred_element_type=jnp.float32)
    contrib = contrib + w_col * bd_ref[0]

    @pl.when(e == 0)
    def _():
        out_ref[...] = contrib

    @pl.when(e != 0)
    def _():
        out_ref[...] += contrib


def _moe_call(hs, ri, rw, wg, wu, bg, bu, wd, bd):
    T, H = hs.shape
    E, _, ED = wg.shape
    grid = (E,)
    return pl.pallas_call(
        _moe_body,
        grid=grid,
        in_specs=[
            pl.BlockSpec((T, H), lambda e: (0, 0)),
            pl.BlockSpec((T, 1), lambda e: (0, 0)),
            pl.BlockSpec((T, E), lambda e: (0, 0)),
            pl.BlockSpec((1, H, ED), lambda e: (e, 0, 0)),
            pl.BlockSpec((1, H, ED), lambda e: (e, 0, 0)),
            pl.BlockSpec((1, 1, ED), lambda e: (e, 0, 0)),
            pl.BlockSpec((1, 1, ED), lambda e: (e, 0, 0)),
            pl.BlockSpec((1, ED, H), lambda e: (e, 0, 0)),
            pl.BlockSpec((1, 1, H), lambda e: (e, 0, 0)),
        ],
        out_specs=pl.BlockSpec((T, H), lambda e: (0, 0)),
        out_shape=jax.ShapeDtypeStruct((T, H), jnp.float32),
    )(hs, ri, rw, wg, wu, bg, bu, wd, bd)


def kernel(hidden_states, router_indices, routing_weights, gate_up_proj,
           gate_up_proj_bias, down_proj, down_proj_bias):
    B, S, H = hidden_states.shape
    E, _, ED2 = gate_up_proj.shape
    ED = ED2 // 2
    T = B * S
    hs = hidden_states.reshape(T, H)
    ri = router_indices.reshape(T, 1).astype(jnp.int32)
    rw = routing_weights.reshape(T, E)
    wg = gate_up_proj[:, :, 0::2]
    wu = gate_up_proj[:, :, 1::2]
    bg = gate_up_proj_bias[:, 0::2].reshape(E, 1, ED)
    bu = gate_up_proj_bias[:, 1::2].reshape(E, 1, ED)
    bd = down_proj_bias.reshape(E, 1, H)
    out = _moe_call(hs, ri, rw, wg, wu, bg, bu, down_proj, bd)
    return out.reshape(B, S, H)
